# Initial kernel scaffold; baseline (speedup 1.0000x reference)
import jax
import jax.numpy as jnp
from jax.experimental import pallas as pl


def kernel(h, edge_index, rel_types, W1, Ws1, b1, W2, Ws2, b2, Wc, bc):
    raise NotImplementedError("write your pallas kernel here")


# speedup vs baseline: 1.8554x; 1.8554x over previous
import functools

import jax
import jax.numpy as jnp
from jax import lax
from jax.experimental import pallas as pl
from jax.experimental.pallas import tpu as pltpu
from jax.experimental.pallas import tpu_sc as plsc

_N = 10000
_R = 14
_H = 128
_E = 320000
_NCLS = 10

_NC = 2
_NS = 16
_NW = _NC * _NS
_C = 128
_NCH = -(-_E // (_NW * _C))
_EPAD = _NW * _NCH * _C
_NPAD = 10112
_RPT = _NPAD // _NS

_BN = 1000


def _sc_edge_agg(table, gidx, dstx):
    mesh = plsc.VectorSubcoreMesh(
        core_axis_name="c", subcore_axis_name="s",
        num_cores=_NC, num_subcores=_NS)

    @functools.partial(
        pl.kernel,
        out_type=jax.ShapeDtypeStruct((_NC, _NPAD, _H), jnp.float32),
        mesh=mesh,
        scratch_types=[
            pltpu.VMEM((_C,), jnp.int32),
            pltpu.VMEM((_C,), jnp.int32),
            pltpu.VMEM((_C, _H), jnp.float32),
            pltpu.VMEM_SHARED((_NPAD, _H), jnp.float32),
            pltpu.SemaphoreType.DMA,
        ],
    )
    def k(table_h, gidx_h, dstx_h, out_h, gi_v, di_v, msg_v, acc_s, sem):
        cid = lax.axis_index("c")
        sid = lax.axis_index("s")
        wid = sid * _NC + cid
        base = sid * _RPT

        zero = jnp.zeros((16,), jnp.float32)

        def zrow(i, carry):
            for c8 in range(_H // 16):
                msg_v[i, pl.ds(c8 * 16, 16)] = zero
            return carry

        lax.fori_loop(0, _C, zrow, 0)
        for kk in range(_RPT // _C):
            pltpu.sync_copy(msg_v, acc_s.at[pl.ds(base + kk * _C, _C)])
        rem = _RPT % _C
        if rem:
            pltpu.sync_copy(msg_v.at[pl.ds(0, rem)],
                            acc_s.at[pl.ds(base + (_RPT // _C) * _C, rem)])
        plsc.subcore_barrier()

        def body(j, carry):
            pltpu.sync_copy(gidx_h.at[wid, j], gi_v)
            pltpu.sync_copy(dstx_h.at[wid, j], di_v)
            pltpu.async_copy(table_h.at[gi_v], msg_v, sem).wait()
            pltpu.sync_copy(msg_v, acc_s.at[di_v], add=True)
            return carry

        lax.fori_loop(0, _NCH, body, 0)
        plsc.subcore_barrier()
        pltpu.sync_copy(acc_s.at[pl.ds(base, _RPT)],
                        out_h.at[cid, pl.ds(base, _RPT)])

    return k(table, gidx, dstx)


def _proj_body(x, W_ref, Ws_ref, b_ref, proj_ref, self_ref):
    for r in range(_R):
        proj_ref[:, r, :] = jnp.dot(x, W_ref[r],
                                    preferred_element_type=jnp.float32)
    self_ref[...] = jnp.dot(x, Ws_ref[...],
                            preferred_element_type=jnp.float32) + b_ref[...]


def _project_l1(h, W, Ws, b):
    def body(h_ref, W_ref, Ws_ref, b_ref, proj_ref, self_ref):
        _proj_body(h_ref[...], W_ref, Ws_ref, b_ref, proj_ref, self_ref)

    return pl.pallas_call(
        body,
        grid=(_N // _BN,),
        in_specs=[
            pl.BlockSpec((_BN, _H), lambda i: (i, 0)),
            pl.BlockSpec((_R, _H, _H), lambda i: (0, 0, 0)),
            pl.BlockSpec((_H, _H), lambda i: (0, 0)),
            pl.BlockSpec((1, _H), lambda i: (0, 0)),
        ],
        out_specs=[
            pl.BlockSpec((_BN, _R, _H), lambda i: (i, 0, 0)),
            pl.BlockSpec((_BN, _H), lambda i: (i, 0)),
        ],
        out_shape=[
            jax.ShapeDtypeStruct((_N, _R, _H), jnp.float32),
            jax.ShapeDtypeStruct((_N, _H), jnp.float32),
        ],
    )(h, W, Ws, b.reshape(1, _H))


def _combine_project_l2(agg, selfp, W, Ws, b):
    def body(agg_ref, self_ref, W_ref, Ws_ref, b_ref, proj_ref, self2_ref):
        x = jnp.maximum(agg_ref[0] + agg_ref[1] + self_ref[...], 0.0)
        _proj_body(x, W_ref, Ws_ref, b_ref, proj_ref, self2_ref)

    return pl.pallas_call(
        body,
        grid=(_N // _BN,),
        in_specs=[
            pl.BlockSpec((2, _BN, _H), lambda i: (0, i, 0)),
            pl.BlockSpec((_BN, _H), lambda i: (i, 0)),
            pl.BlockSpec((_R, _H, _H), lambda i: (0, 0, 0)),
            pl.BlockSpec((_H, _H), lambda i: (0, 0)),
            pl.BlockSpec((1, _H), lambda i: (0, 0)),
        ],
        out_specs=[
            pl.BlockSpec((_BN, _R, _H), lambda i: (i, 0, 0)),
            pl.BlockSpec((_BN, _H), lambda i: (i, 0)),
        ],
        out_shape=[
            jax.ShapeDtypeStruct((_N, _R, _H), jnp.float32),
            jax.ShapeDtypeStruct((_N, _H), jnp.float32),
        ],
    )(agg, selfp, W, Ws, b.reshape(1, _H))


def _head(agg, selfp, Wc, bc):
    def body(agg_ref, self_ref, Wc_ref, bc_ref, out_ref, acc_ref):
        i = pl.program_id(0)
        x = jnp.maximum(agg_ref[0] + agg_ref[1] + self_ref[...], 0.0)
        part = jnp.sum(x, axis=0, keepdims=True)

        @pl.when(i == 0)
        def _():
            acc_ref[...] = part

        @pl.when(i > 0)
        def _():
            acc_ref[...] = acc_ref[...] + part

        @pl.when(i == pl.num_programs(0) - 1)
        def _():
            hg = acc_ref[...] * (1.0 / _N)
            logits = jnp.dot(hg, Wc_ref[...],
                             preferred_element_type=jnp.float32) + bc_ref[...]
            m = jnp.max(logits, axis=1, keepdims=True)
            e = jnp.exp(logits - m)
            out_ref[...] = e / jnp.sum(e, axis=1, keepdims=True)

    return pl.pallas_call(
        body,
        grid=(_N // _BN,),
        in_specs=[
            pl.BlockSpec((2, _BN, _H), lambda i: (0, i, 0)),
            pl.BlockSpec((_BN, _H), lambda i: (i, 0)),
            pl.BlockSpec((_H, _NCLS), lambda i: (0, 0)),
            pl.BlockSpec((1, _NCLS), lambda i: (0, 0)),
        ],
        out_specs=pl.BlockSpec((1, _NCLS), lambda i: (0, 0)),
        out_shape=jax.ShapeDtypeStruct((1, _NCLS), jnp.float32),
        scratch_shapes=[pltpu.VMEM((1, _H), jnp.float32)],
    )(agg, selfp, Wc, bc.reshape(1, _NCLS))


def kernel(h, edge_index, rel_types, W1, Ws1, b1, W2, Ws2, b2, Wc, bc):
    h = h.astype(jnp.float32)
    src = edge_index[0]
    dst = edge_index[1]

    pad = _EPAD - _E
    gidx = src * _R + rel_types
    gidx = jnp.concatenate([gidx, jnp.zeros((pad,), jnp.int32)])
    gidx = gidx.reshape(_NW, _NCH, _C)
    dstx = jnp.concatenate([dst, jnp.full((pad,), _N, jnp.int32)])
    dstx = dstx.reshape(_NW, _NCH, _C)

    proj1, self1 = _project_l1(h, W1, Ws1, b1)
    agg1 = _sc_edge_agg(proj1.reshape(_N * _R, _H), gidx, dstx)
    proj2, self2 = _combine_project_l2(agg1, self1, W2, Ws2, b2)
    agg2 = _sc_edge_agg(proj2.reshape(_N * _R, _H), gidx, dstx)
    return _head(agg2, self2, Wc, bc)
